# baseline (speedup 1.0000x reference)
import jax
import jax.numpy as jnp
from jax.experimental import pallas as pl

_BLOCK = 2000


def _cell_kernel(x_ref, h_ref, c_ref, w_ref, th_ref, bias_ref, wc_ref,
                 lin_w_ref, lin_b_ref, out_ref, hout_ref, cout_ref):
    xb = x_ref[...]
    hb = h_ref[...]
    cb = c_ref[...]
    g = (jnp.dot(xb, w_ref[...], preferred_element_type=jnp.float32)
         + jnp.dot(hb, th_ref[...], preferred_element_type=jnp.float32)
         + bias_ref[...])
    wc = wc_ref[...]
    gi = jax.nn.sigmoid(g[:, 0:3] + wc[0:1, :] * cb)
    gf = jax.nn.sigmoid(g[:, 3:6] + wc[1:2, :] * cb)
    gt = jnp.tanh(g[:, 6:9])
    c_new = gf * cb + gi * gt
    go = jax.nn.sigmoid(g[:, 9:12] + wc[2:3, :] * c_new)
    h_new = go * jnp.tanh(c_new)
    out_ref[...] = (jnp.sum(jax.nn.relu(h_new) * lin_w_ref[...],
                            axis=1, keepdims=True) + lin_b_ref[...])
    hout_ref[...] = h_new
    cout_ref[...] = c_new


def kernel(x, edge_index, edge_weight, h, c,
           W_i, W_f, W_c, W_o,
           Th_i, Th_f, Th_c, Th_o,
           cb_i, cb_f, cb_c, cb_o,
           b_i, b_f, b_c, b_o,
           wc_i, wc_f, wc_o,
           lin_W, lin_b):
    n, in_ch = x.shape
    out_ch = h.shape[1]
    w_all = jnp.concatenate([W_i, W_f, W_c, W_o], axis=1)
    th_all = jnp.concatenate([Th_i, Th_f, Th_c, Th_o], axis=1)
    bias_all = jnp.concatenate(
        [cb_i[None, :] + b_i, cb_f[None, :] + b_f,
         cb_c[None, :] + b_c, cb_o[None, :] + b_o], axis=1)
    wc_all = jnp.concatenate([wc_i, wc_f, wc_o], axis=0)
    lin_b2 = lin_b.reshape(1, 1)

    block = _BLOCK
    grid = (n // block,)
    row_spec = lambda w: pl.BlockSpec((block, w), lambda i: (i, 0))
    full_spec = lambda a: pl.BlockSpec(a.shape, lambda i: (0, 0))

    out, h_new, c_new = pl.pallas_call(
        _cell_kernel,
        grid=grid,
        in_specs=[
            row_spec(in_ch), row_spec(out_ch), row_spec(out_ch),
            full_spec(w_all), full_spec(th_all), full_spec(bias_all),
            full_spec(wc_all), full_spec(lin_W), full_spec(lin_b2),
        ],
        out_specs=[row_spec(1), row_spec(out_ch), row_spec(out_ch)],
        out_shape=[
            jax.ShapeDtypeStruct((n, 1), jnp.float32),
            jax.ShapeDtypeStruct((n, out_ch), jnp.float32),
            jax.ShapeDtypeStruct((n, out_ch), jnp.float32),
        ],
    )(x, h, c, w_all, th_all, bias_all, wc_all, lin_W, lin_b2)
    return (out, h_new, c_new)
